# baseline (speedup 1.0000x reference)
import functools

import jax
import jax.numpy as jnp
from jax import lax
from jax.experimental import pallas as pl
from jax.experimental.pallas import tpu as pltpu
from jax.experimental.pallas import tpu_sc as plsc


def kernel(rel_pos_emb, seq_len):
    del seq_len
    V, D = rel_pos_emb.shape
    N = (V + 1) // 2

    info = plsc.get_sparse_core_info()
    NC, NS = info.num_cores, info.num_subcores
    NW = NC * NS
    rpw = N // NW
    JC = N // 4
    win = JC + 8 * (rpw - 1)
    NQ = N // JC

    mesh = plsc.VectorSubcoreMesh(core_axis_name="c", subcore_axis_name="s")

    @functools.partial(
        pl.kernel,
        mesh=mesh,
        out_type=jax.ShapeDtypeStruct((N, N, D), jnp.float32),
        scratch_types=[
            pltpu.VMEM((2, win, D), jnp.float32),
            pltpu.SemaphoreType.DMA,
            pltpu.SemaphoreType.DMA,
        ],
    )
    def sliding_copy(t8_hbm, out_hbm, buf, lsem, ssem):
        c = lax.axis_index("c")
        s = lax.axis_index("s")
        wid = s * NC + c
        rcls = wid % 8
        g = wid // 8
        i_base = 7 - rcls + 8 * rpw * g
        b_last = N - 8 - 8 * (rpw * g + rpw - 1)

        def load(q):
            base = pl.multiple_of(b_last + q * JC, 8)
            return pltpu.async_copy(
                t8_hbm.at[rcls, pl.ds(base, win)], buf.at[q % 2], lsem
            )

        loads = {0: load(0)}
        stores = {}
        for q in range(NQ):
            if q >= 2:
                for cp in stores[q - 2]:
                    cp.wait()
                loads[q] = load(q)
            loads[q].wait()
            sts = []
            for m in range(rpw):
                off = 8 * (rpw - 1 - m)
                sts.append(
                    pltpu.async_copy(
                        buf.at[q % 2, pl.ds(off, JC)],
                        out_hbm.at[i_base + 8 * m, pl.ds(q * JC, JC)],
                        ssem,
                    )
                )
            stores[q] = sts
            if q == 0:
                loads[1] = load(1)
        for q in (NQ - 2, NQ - 1):
            for cp in stores[q]:
                cp.wait()

    pad = jnp.concatenate(
        [rel_pos_emb, jnp.broadcast_to(rel_pos_emb[-1:], (2 * N + 7 - V, D))]
    )
    t8 = jnp.stack([lax.slice_in_dim(pad, s, s + 2 * N) for s in range(8)])
    return sliding_copy(t8)


# speedup vs baseline: 1.0601x; 1.0601x over previous
import functools

import jax
import jax.numpy as jnp
from jax import lax
from jax.experimental import pallas as pl
from jax.experimental.pallas import tpu as pltpu
from jax.experimental.pallas import tpu_sc as plsc


def kernel(rel_pos_emb, seq_len):
    del seq_len
    V, D = rel_pos_emb.shape
    N = (V + 1) // 2

    info = plsc.get_sparse_core_info()
    NC, NS = info.num_cores, info.num_subcores
    NW = NC * NS
    rpw = N // NW
    JC = N // 2
    win = JC + 8 * (rpw - 1)

    mesh = plsc.VectorSubcoreMesh(core_axis_name="c", subcore_axis_name="s")

    @functools.partial(
        pl.kernel,
        mesh=mesh,
        out_type=jax.ShapeDtypeStruct((N, N, D), jnp.float32),
        scratch_types=[
            pltpu.VMEM((win, D), jnp.float32),
            pltpu.SemaphoreType.DMA,
        ],
    )
    def sliding_copy(t8_hbm, out_hbm, buf, sem):
        c = lax.axis_index("c")
        s = lax.axis_index("s")
        wid = s * NC + c
        rcls = wid % 8
        g = wid // 8
        i_base = 7 - rcls + 8 * rpw * g
        b_last = N - 8 - 8 * (rpw * g + rpw - 1)
        for j0 in (0, JC):
            base = pl.multiple_of(b_last + j0, 8)
            pltpu.sync_copy(t8_hbm.at[rcls, pl.ds(base, win)], buf)
            copies = []
            for m in range(rpw):
                off = 8 * (rpw - 1 - m)
                copies.append(
                    pltpu.async_copy(
                        buf.at[pl.ds(off, JC)],
                        out_hbm.at[i_base + 8 * m, pl.ds(j0, JC)],
                        sem,
                    )
                )
            for cp in copies:
                cp.wait()

    pad = jnp.concatenate(
        [rel_pos_emb, jnp.broadcast_to(rel_pos_emb[-1:], (2 * N + 7 - V, D))]
    )
    t8 = jnp.concatenate(
        [lax.slice_in_dim(pad, s, s + 2 * N) for s in range(8)], axis=0
    ).reshape(8, 2 * N, D)
    return sliding_copy(t8)
